# baseline (speedup 1.0000x reference)
import jax
import jax.numpy as jnp
from jax.experimental import pallas as pl
from jax.experimental.pallas import tpu as pltpu

D = 2048
BATCH = 16384
BM = 1024


def _table_kernel(tt_ref, w1_ref, b_ref, t_ref):
    t_ref[...] = jax.lax.dot_general(
        tt_ref[...], w1_ref[...], (((1,), (1,)), ((), ())),
        preferred_element_type=jnp.float32) + b_ref[...]


def _main_kernel(emb_ref, idx_ref, t_ref, w2_ref, out_ref):
    x = emb_ref[...].astype(jnp.bfloat16)
    acc = jax.lax.dot_general(
        x, w2_ref[...], (((1,), (1,)), ((), ())),
        preferred_element_type=jnp.float32)
    idx = idx_ref[...]
    t = t_ref[...]
    addend = jnp.where(
        idx == 0, t[0:1],
        jnp.where(idx == 1, t[1:2],
                  jnp.where(idx == 2, t[2:3], t[3:4])))
    out_ref[...] = acc + addend


def kernel(embedding, task_idxs, task_table, W, b):
    n = W.shape[0]
    t = pl.pallas_call(
        _table_kernel,
        grid=(1,),
        in_specs=[
            pl.BlockSpec(task_table.shape, lambda i: (0, 0)),
            pl.BlockSpec((n, D), lambda i: (0, 0)),
            pl.BlockSpec((1, n), lambda i: (0, 0)),
        ],
        out_specs=pl.BlockSpec((task_table.shape[0], n), lambda i: (0, 0)),
        out_shape=jax.ShapeDtypeStruct((task_table.shape[0], n), jnp.float32),
    )(task_table, W, b.reshape(1, n))

    idx2d = task_idxs.astype(jnp.int32).reshape(BATCH, 1)

    grid = (BATCH // BM,)
    out = pl.pallas_call(
        _main_kernel,
        grid=grid,
        in_specs=[
            pl.BlockSpec((BM, D), lambda i: (i, 0)),
            pl.BlockSpec((BM, 1), lambda i: (i, 0)),
            pl.BlockSpec(t.shape, lambda i: (0, 0)),
            pl.BlockSpec((n, D), lambda i: (0, 0)),
        ],
        out_specs=pl.BlockSpec((BM, n), lambda i: (i, 0)),
        out_shape=jax.ShapeDtypeStruct((BATCH, n), jnp.float32),
    )(embedding, idx2d, t, W[:, D:].astype(jnp.bfloat16))
    return out


# speedup vs baseline: 1.0017x; 1.0017x over previous
import jax
import jax.numpy as jnp
from jax.experimental import pallas as pl
from jax.experimental.pallas import tpu as pltpu

D = 2048
BATCH = 16384
BM = 1024


def _table_kernel(tt_ref, w1_ref, b_ref, t_ref):
    t_ref[...] = jax.lax.dot_general(
        tt_ref[...], w1_ref[...], (((1,), (1,)), ((), ())),
        preferred_element_type=jnp.float32) + b_ref[...]


def _main_kernel(emb_ref, idx_ref, t_ref, w2_ref, out_ref):
    x = emb_ref[...].astype(jnp.bfloat16)
    acc = jax.lax.dot_general(
        x, w2_ref[...], (((1,), (1,)), ((), ())),
        preferred_element_type=jnp.float32)
    idx = idx_ref[...]
    t = t_ref[...]
    addend = jnp.where(
        idx == 0, t[0:1],
        jnp.where(idx == 1, t[1:2],
                  jnp.where(idx == 2, t[2:3], t[3:4])))
    out_ref[...] = acc + addend


def kernel(embedding, task_idxs, task_table, W, b):
    n = W.shape[0]
    t = pl.pallas_call(
        _table_kernel,
        grid=(1,),
        in_specs=[
            pl.BlockSpec(task_table.shape, lambda i: (0, 0)),
            pl.BlockSpec((n, D), lambda i: (0, 0)),
            pl.BlockSpec((1, n), lambda i: (0, 0)),
        ],
        out_specs=pl.BlockSpec((task_table.shape[0], n), lambda i: (0, 0)),
        out_shape=jax.ShapeDtypeStruct((task_table.shape[0], n), jnp.float32),
    )(task_table, W, b.reshape(1, n))

    idx2d = task_idxs.astype(jnp.int32).reshape(BATCH, 1)

    grid = (BATCH // BM,)
    out = pl.pallas_call(
        _main_kernel,
        grid=grid,
        in_specs=[
            pl.BlockSpec((BM, D), lambda i: (i, 0)),
            pl.BlockSpec((BM, 1), lambda i: (i, 0)),
            pl.BlockSpec(t.shape, lambda i: (0, 0)),
            pl.BlockSpec((n, D), lambda i: (0, 0)),
        ],
        out_specs=pl.BlockSpec((BM, n), lambda i: (i, 0)),
        out_shape=jax.ShapeDtypeStruct((BATCH, n), jnp.float32),
        compiler_params=pltpu.CompilerParams(
            dimension_semantics=("parallel",)),
    )(embedding, idx2d, t, W[:, D:].astype(jnp.bfloat16))
    return out


# speedup vs baseline: 1.0490x; 1.0472x over previous
import jax
import jax.numpy as jnp
from jax.experimental import pallas as pl
from jax.experimental.pallas import tpu as pltpu

D = 2048
BATCH = 16384
BM = 1024


def _table_kernel(tt_ref, w1_ref, b_ref, t_ref):
    t_ref[...] = jax.lax.dot_general(
        tt_ref[...], w1_ref[...], (((1,), (1,)), ((), ())),
        preferred_element_type=jnp.float32) + b_ref[...]


def _main_kernel(emb_ref, idx_ref, t_ref, w2_ref, out_ref):
    acc = jax.lax.dot_general(
        emb_ref[...], w2_ref[...], (((1,), (1,)), ((), ())),
        preferred_element_type=jnp.float32)
    idx = idx_ref[...]
    t = t_ref[...]
    addend = jnp.where(
        idx == 0, t[0:1],
        jnp.where(idx == 1, t[1:2],
                  jnp.where(idx == 2, t[2:3], t[3:4])))
    out_ref[...] = acc + addend


def kernel(embedding, task_idxs, task_table, W, b):
    n = W.shape[0]
    t = pl.pallas_call(
        _table_kernel,
        grid=(1,),
        in_specs=[
            pl.BlockSpec(task_table.shape, lambda i: (0, 0)),
            pl.BlockSpec((n, D), lambda i: (0, 0)),
            pl.BlockSpec((1, n), lambda i: (0, 0)),
        ],
        out_specs=pl.BlockSpec((task_table.shape[0], n), lambda i: (0, 0)),
        out_shape=jax.ShapeDtypeStruct((task_table.shape[0], n), jnp.float32),
    )(task_table, W, b.reshape(1, n))

    idx2d = task_idxs.astype(jnp.int32).reshape(BATCH, 1)

    grid = (BATCH // BM,)
    out = pl.pallas_call(
        _main_kernel,
        grid=grid,
        in_specs=[
            pl.BlockSpec((BM, D), lambda i: (i, 0)),
            pl.BlockSpec((BM, 1), lambda i: (i, 0)),
            pl.BlockSpec(t.shape, lambda i: (0, 0)),
            pl.BlockSpec((n, D), lambda i: (0, 1)),
        ],
        out_specs=pl.BlockSpec((BM, n), lambda i: (i, 0)),
        out_shape=jax.ShapeDtypeStruct((BATCH, n), jnp.float32),
        compiler_params=pltpu.CompilerParams(
            dimension_semantics=("parallel",)),
    )(embedding, idx2d, t, W)
    return out
